# baseline (speedup 1.0000x reference)
import functools

import jax
import jax.numpy as jnp
from jax.experimental import pallas as pl
from jax.experimental.pallas import tpu as pltpu

_TB = 256
_D = 1024
_E1 = 1024
_E2 = 2048


def _moe_block(x_ref, gw_ref, w1i_ref, w1o_ref, w2i_ref, w2o_ref,
               out_ref, logits_ref):
    x = x_ref[...]

    lp = jnp.dot(x, gw_ref[...], preferred_element_type=jnp.float32)
    logits_ref[...] = lp[:, :3]
    l0, l1, l2 = lp[:, 0:1], lp[:, 1:2], lp[:, 2:3]
    m = jnp.maximum(jnp.maximum(l0, l1), l2)
    e0 = jnp.exp(l0 - m)
    e1 = jnp.exp(l1 - m)
    e2 = jnp.exp(l2 - m)
    s = e0 + e1 + e2
    p0, p1, p2 = e0 / s, e1 / s, e2 / s
    pmin = jnp.minimum(jnp.minimum(p0, p1), p2)
    drop2 = p2 <= pmin
    drop1 = jnp.logical_and(jnp.logical_not(drop2), p1 <= pmin)
    drop0 = jnp.logical_not(jnp.logical_or(drop1, drop2))
    w0 = jnp.where(drop0, 0.0, p0)
    w1 = jnp.where(drop1, 0.0, p1)
    w2 = jnp.where(drop2, 0.0, p2)
    inv = 1.0 / (w0 + w1 + w2)
    w0i, w1i, w2i = w0 * inv, w1 * inv, w2 * inv

    x1 = w1i * x
    x2 = w2i * x

    acc = w0i * x
    a = jnp.dot(x, w1i_ref[:, :_E1], preferred_element_type=jnp.float32)
    b = jnp.dot(x1, w1i_ref[:, _E1:], preferred_element_type=jnp.float32)
    g = ((a * b) / (1.0 + jnp.exp(-a))).astype(jnp.bfloat16)
    acc = acc + jnp.dot(g, w1o_ref[...], preferred_element_type=jnp.float32)
    for c in range(2):
        lo = c * 1024
        a = jnp.dot(x, w2i_ref[:, lo:lo + 1024],
                    preferred_element_type=jnp.float32)
        b = jnp.dot(x2, w2i_ref[:, _E2 + lo:_E2 + lo + 1024],
                    preferred_element_type=jnp.float32)
        g = ((a * b) / (1.0 + jnp.exp(-a))).astype(jnp.bfloat16)
        acc = acc + jnp.dot(g, w2o_ref[lo:lo + 1024, :],
                            preferred_element_type=jnp.float32)
    out_ref[...] = acc


@functools.partial(jax.jit, static_argnums=())
def kernel(hidden_states, output_expert_usage_loss, pad_mask, gate_w,
           w1_in, w1_out, w2_in, w2_out):
    B, S, D = hidden_states.shape
    T = B * S
    h = hidden_states.reshape(T, D)
    gw = jnp.zeros((D, 128), gate_w.dtype).at[:, :3].set(gate_w)

    grid = (T // _TB,)
    full = lambda i: (0, 0)
    out, logits = pl.pallas_call(
        _moe_block,
        grid=grid,
        in_specs=[
            pl.BlockSpec((_TB, D), lambda i: (i, 0)),
            pl.BlockSpec((D, 128), full),
            pl.BlockSpec((D, 2 * _E1), full),
            pl.BlockSpec((_E1, D), full),
            pl.BlockSpec((D, 2 * _E2), full),
            pl.BlockSpec((_E2, D), full),
        ],
        out_specs=[
            pl.BlockSpec((_TB, D), lambda i: (i, 0)),
            pl.BlockSpec((_TB, 3), lambda i: (i, 0)),
        ],
        out_shape=[
            jax.ShapeDtypeStruct((T, D), jnp.float32),
            jax.ShapeDtypeStruct((T, 3), jnp.float32),
        ],
        compiler_params=pltpu.CompilerParams(
            dimension_semantics=("arbitrary",),
            vmem_limit_bytes=100 * 1024 * 1024,
        ),
    )(h, gw, w1_in, w1_out, w2_in, w2_out)

    return out.reshape(B, S, D), logits


# speedup vs baseline: 1.0357x; 1.0357x over previous
import functools

import jax
import jax.numpy as jnp
from jax.experimental import pallas as pl
from jax.experimental.pallas import tpu as pltpu

_TB = 512
_D = 1024
_E1 = 1024
_E2 = 2048


def _moe_block(x_ref, gw_ref, w1i_ref, w1o_ref, w2i_ref, w2o_ref,
               out_ref, logits_ref):
    x = x_ref[...]

    lp = jnp.dot(x, gw_ref[...], preferred_element_type=jnp.float32)
    logits_ref[...] = lp[:, :3]
    l0, l1, l2 = lp[:, 0:1], lp[:, 1:2], lp[:, 2:3]
    m = jnp.maximum(jnp.maximum(l0, l1), l2)
    e0 = jnp.exp(l0 - m)
    e1 = jnp.exp(l1 - m)
    e2 = jnp.exp(l2 - m)
    s = e0 + e1 + e2
    p0, p1, p2 = e0 / s, e1 / s, e2 / s
    pmin = jnp.minimum(jnp.minimum(p0, p1), p2)
    drop2 = p2 <= pmin
    drop1 = jnp.logical_and(jnp.logical_not(drop2), p1 <= pmin)
    drop0 = jnp.logical_not(jnp.logical_or(drop1, drop2))
    w0 = jnp.where(drop0, 0.0, p0)
    w1 = jnp.where(drop1, 0.0, p1)
    w2 = jnp.where(drop2, 0.0, p2)
    inv = 1.0 / (w0 + w1 + w2)
    w0i, w1i, w2i = w0 * inv, w1 * inv, w2 * inv

    x1 = w1i * x
    x2 = w2i * x

    acc = w0i * x
    a = jnp.dot(x, w1i_ref[:, :_E1], preferred_element_type=jnp.float32)
    b = jnp.dot(x1, w1i_ref[:, _E1:], preferred_element_type=jnp.float32)
    g = ((a * b) / (1.0 + jnp.exp(-a))).astype(jnp.bfloat16)
    acc = acc + jnp.dot(g, w1o_ref[...], preferred_element_type=jnp.float32)
    for c in range(2):
        lo = c * 1024
        a = jnp.dot(x, w2i_ref[:, lo:lo + 1024],
                    preferred_element_type=jnp.float32)
        b = jnp.dot(x2, w2i_ref[:, _E2 + lo:_E2 + lo + 1024],
                    preferred_element_type=jnp.float32)
        g = ((a * b) / (1.0 + jnp.exp(-a))).astype(jnp.bfloat16)
        acc = acc + jnp.dot(g, w2o_ref[lo:lo + 1024, :],
                            preferred_element_type=jnp.float32)
    out_ref[...] = acc


@functools.partial(jax.jit, static_argnums=())
def kernel(hidden_states, output_expert_usage_loss, pad_mask, gate_w,
           w1_in, w1_out, w2_in, w2_out):
    B, S, D = hidden_states.shape
    T = B * S
    h = hidden_states.reshape(T, D)
    gw = jnp.zeros((D, 128), gate_w.dtype).at[:, :3].set(gate_w)

    grid = (T // _TB,)
    full = lambda i: (0, 0)
    out, logits = pl.pallas_call(
        _moe_block,
        grid=grid,
        in_specs=[
            pl.BlockSpec((_TB, D), lambda i: (i, 0)),
            pl.BlockSpec((D, 128), full),
            pl.BlockSpec((D, 2 * _E1), full),
            pl.BlockSpec((_E1, D), full),
            pl.BlockSpec((D, 2 * _E2), full),
            pl.BlockSpec((_E2, D), full),
        ],
        out_specs=[
            pl.BlockSpec((_TB, D), lambda i: (i, 0)),
            pl.BlockSpec((_TB, 3), lambda i: (i, 0)),
        ],
        out_shape=[
            jax.ShapeDtypeStruct((T, D), jnp.float32),
            jax.ShapeDtypeStruct((T, 3), jnp.float32),
        ],
        compiler_params=pltpu.CompilerParams(
            dimension_semantics=("arbitrary",),
            vmem_limit_bytes=100 * 1024 * 1024,
        ),
    )(h, gw, w1_in, w1_out, w2_in, w2_out)

    return out.reshape(B, S, D), logits
